# baseline (speedup 1.0000x reference)
import jax
import jax.numpy as jnp
from jax import lax
from jax.experimental import pallas as pl
from jax.experimental.pallas import tpu as pltpu
from jax.experimental.pallas import tpu_sc as plsc

NC = 2
NS = 16
L = 16
NW = NC * NS

B = 16384
D = 64
H = 256
BPW = B // NW

CH = 256
NCHUNK = BPW // CH

BLK = 2048
GRID = B // BLK


def _gather_rowdma_body(idx_hbm, tab_hbm, out_hbm, idx_v, rows_v, sem):
    wid = lax.axis_index("s") * NC + lax.axis_index("c")
    base = wid * BPW
    pltpu.sync_copy(idx_hbm.at[pl.ds(base, BPW)], idx_v)

    def chunk(c, carry0):
        cbase = c * CH

        def issue16(k, carry):
            vec = idx_v[pl.ds(cbase + k * L, L)]
            for j in range(L):
                u = vec[j]
                pltpu.async_copy(tab_hbm.at[pl.ds(u, 1)],
                                 rows_v.at[pl.ds(k * L + j, 1)], sem)
            return carry

        lax.fori_loop(0, CH // L, issue16, 0)
        pltpu.make_async_copy(tab_hbm.at[pl.ds(0, CH)], rows_v, sem).wait()

        def row(i, carry):
            pltpu.async_copy(rows_v.at[pl.ds(i, 1)],
                             out_hbm.at[pl.ds(base + cbase + i, 1)], sem)
            return carry

        lax.fori_loop(0, CH, row, 0)
        pltpu.make_async_copy(out_hbm.at[pl.ds(0, CH)], rows_v, sem).wait()
        return carry0

    lax.fori_loop(0, NCHUNK, chunk, 0)


_gather_rowdma = pl.kernel(
    _gather_rowdma_body,
    mesh=plsc.VectorSubcoreMesh(core_axis_name="c", subcore_axis_name="s"),
    out_type=jax.ShapeDtypeStruct((B, D), jnp.float32),
    scratch_types=[
        pltpu.VMEM((BPW,), jnp.int32),
        pltpu.VMEM((CH, D), jnp.float32),
        pltpu.SemaphoreType.DMA,
    ],
)


def _gather_stream_body(idx_hbm, tab_hbm, out_hbm, idx_v, rows_v, sem):
    wid = lax.axis_index("s") * NC + lax.axis_index("c")
    base = wid * BPW
    pltpu.sync_copy(idx_hbm.at[pl.ds(base, BPW)], idx_v)
    pltpu.async_copy(tab_hbm.at[idx_v], rows_v, sem).wait()
    pltpu.sync_copy(rows_v, out_hbm.at[pl.ds(base, BPW)])


_gather_stream = pl.kernel(
    _gather_stream_body,
    mesh=plsc.VectorSubcoreMesh(core_axis_name="c", subcore_axis_name="s"),
    compiler_params=pltpu.CompilerParams(use_tc_tiling_on_sc=False),
    out_type=jax.ShapeDtypeStruct((B, D), jnp.float32),
    scratch_types=[
        pltpu.VMEM((BPW,), jnp.int32),
        pltpu.VMEM((BPW, D), jnp.float32),
        pltpu.SemaphoreType.DMA,
    ],
)


def _mlp_body(ue_ref, ie_ref, w1_ref, b1_ref, w2_ref, b2_ref, out_ref):
    mix = ue_ref[...] * ie_ref[...]
    h = jnp.dot(mix, w1_ref[...], preferred_element_type=jnp.float32)
    h = jnp.maximum(h + b1_ref[...], 0.0)
    z = jnp.dot(h, w2_ref[...], preferred_element_type=jnp.float32)
    out_ref[...] = jax.nn.sigmoid(z + b2_ref[...])


def _mlp(ue, ie, W1, b1, W2, b2):
    return pl.pallas_call(
        _mlp_body,
        grid=(GRID,),
        in_specs=[
            pl.BlockSpec((BLK, D), lambda i: (i, 0)),
            pl.BlockSpec((BLK, D), lambda i: (i, 0)),
            pl.BlockSpec((D, H), lambda i: (0, 0)),
            pl.BlockSpec((1, H), lambda i: (0, 0)),
            pl.BlockSpec((H, 1), lambda i: (0, 0)),
            pl.BlockSpec((1, 1), lambda i: (0, 0)),
        ],
        out_specs=pl.BlockSpec((BLK, 1), lambda i: (i, 0)),
        out_shape=jax.ShapeDtypeStruct((B, 1), jnp.float32),
    )(ue, ie, W1, b1.reshape(1, H), W2, b2.reshape(1, 1))


def kernel(user, item, user_table, item_table, W1, b1, W2, b2):
    user = user.astype(jnp.int32)
    item = item.astype(jnp.int32)
    ue = _gather_rowdma(user, user_table)
    ie = _gather_stream(item, item_table)
    out = _mlp(ue, ie, W1, b1, W2, b2)
    return out.reshape(-1)
